# baseline (speedup 1.0000x reference)
import functools

import jax
import jax.numpy as jnp
from jax import lax
from jax.experimental import pallas as pl
from jax.experimental.pallas import tpu as pltpu
from jax.experimental.pallas import tpu_sc as plsc

_DIM = 2048
_C = 7
_SEG = _DIM // _C
_ROWS = 16384
_NC = 2
_NS = 16
_NW = _NC * _NS

_SC_ROWS = 6144
_RPT = _SC_ROWS // _NW
_BLK = 512
_NB = _ROWS // _BLK
_TC_OFF = _SC_ROWS // _BLK
_NBT = _NB - _TC_OFF

_mesh = plsc.VectorSubcoreMesh(core_axis_name="c", subcore_axis_name="s")


def _alloc_body(dummy_ref, big_ref):
    dummy_ref[...] = jnp.zeros_like(dummy_ref)


def _alloc_out():
    dummy, big = pl.pallas_call(
        _alloc_body,
        out_shape=(
            jax.ShapeDtypeStruct((8, 128), jnp.float32),
            jax.ShapeDtypeStruct((_ROWS, _DIM), jnp.float32),
        ),
        out_specs=(
            pl.BlockSpec(memory_space=pltpu.VMEM),
            pl.BlockSpec(memory_space=pl.ANY),
        ),
    )()
    return big


@functools.partial(
    pl.kernel,
    mesh=_mesh,
    out_type=jax.ShapeDtypeStruct((16,), jnp.int32),
    scratch_types=[
        pltpu.VMEM((_RPT,), jnp.int32),
        pltpu.VMEM((8, _DIM), jnp.float32),
        pltpu.SemaphoreType.DMA,
        pltpu.SemaphoreType.DMA,
    ],
)
def _sc_writer(lab_hbm, tab_hbm, out_hbm, done_hbm, lab_v, tab_v, sem_in, sem_out):
    wid = lax.axis_index("s") * _NC + lax.axis_index("c")
    base = wid * _RPT
    pltpu.async_copy(lab_hbm.at[pl.ds(base, _RPT)], lab_v, sem_in).wait()
    pltpu.async_copy(tab_hbm, tab_v, sem_in).wait()

    @pl.loop(0, _RPT, step=16)
    def _fire(i):
        labs = lab_v[pl.ds(i, 16)]
        for k in range(16):
            pltpu.make_async_copy(
                tab_v.at[labs[k]], out_hbm.at[base + i + k], sem_out
            ).start()

    @pl.loop(0, _RPT)
    def _drain(i):
        pltpu.make_async_copy(
            tab_hbm.at[0], out_hbm.at[base + i], sem_out
        ).wait()

    @pl.when(wid == 0)
    def _():
        pltpu.async_copy(lab_v.at[pl.ds(0, 16)], done_hbm, sem_in).wait()


def _mask_block(labs_row):
    lab = labs_row.reshape(_BLK, 1)
    start = lab * _SEG
    end = jnp.where(lab == _C - 1, _DIM, start + _SEG)
    j = jax.lax.broadcasted_iota(jnp.int32, (_BLK, _DIM), 1)
    return ((j >= start) & (j < end)).astype(jnp.float32)


_NG = _NBT // 2
_OFF2 = _TC_OFF // 2


def _tc_writer_body(lab_ref, out_hbm, done_ref, buf, sems):
    g = pl.program_id(0)
    done_ref[...] = jnp.zeros_like(done_ref)
    row0 = (_TC_OFF + 2 * g) * _BLK

    @pl.when(g > 0)
    def _():
        pltpu.make_async_copy(
            buf.at[0], out_hbm.at[pl.ds(row0 - 2 * _BLK, _BLK)], sems.at[0]
        ).wait()

    buf[0, :, :] = _mask_block(lab_ref[0, 0, :])
    cp0 = pltpu.make_async_copy(
        buf.at[0], out_hbm.at[pl.ds(row0, _BLK)], sems.at[0]
    )
    cp0.start()

    @pl.when(g > 0)
    def _():
        pltpu.make_async_copy(
            buf.at[1], out_hbm.at[pl.ds(row0 - _BLK, _BLK)], sems.at[1]
        ).wait()

    buf[1, :, :] = _mask_block(lab_ref[0, 1, :])
    cp1 = pltpu.make_async_copy(
        buf.at[1], out_hbm.at[pl.ds(row0 + _BLK, _BLK)], sems.at[1]
    )
    cp1.start()

    @pl.when(g == _NG - 1)
    def _():
        cp0.wait()
        cp1.wait()


def _tc_writer(labs2g, out_buf):
    return pl.pallas_call(
        _tc_writer_body,
        grid=(_NG,),
        in_specs=[
            pl.BlockSpec((1, 2, _BLK), lambda g: (_OFF2 + g, 0, 0)),
            pl.BlockSpec(memory_space=pl.ANY),
        ],
        out_shape=jax.ShapeDtypeStruct((8, 128), jnp.float32),
        out_specs=pl.BlockSpec(memory_space=pltpu.VMEM),
        scratch_shapes=[
            pltpu.VMEM((2, _BLK, _DIM), jnp.float32),
            pltpu.SemaphoreType.DMA((2,)),
        ],
    )(labs2g, out_buf)


def _final_body(big_ref, d1_ref, d2_ref, out_ref):
    pass


def _finalize(big, d1, d2):
    return pl.pallas_call(
        _final_body,
        in_specs=[
            pl.BlockSpec(memory_space=pl.ANY),
            pl.BlockSpec(memory_space=pltpu.VMEM),
            pl.BlockSpec(memory_space=pltpu.SMEM),
        ],
        out_shape=jax.ShapeDtypeStruct((_ROWS, _DIM), jnp.float32),
        out_specs=pl.BlockSpec(memory_space=pl.ANY),
        input_output_aliases={0: 0},
    )(big, d1, d2)


def kernel(inputs_label):
    r = jnp.arange(8, dtype=jnp.int32)[:, None]
    j = jnp.arange(_DIM, dtype=jnp.int32)[None, :]
    start = r * _SEG
    end = jnp.where(r >= _C - 1, _DIM, start + _SEG)
    tab = ((j >= start) & (j < end)).astype(jnp.float32)

    out_buf = _alloc_out()
    d2 = _sc_writer(inputs_label, tab, out_buf)
    labs2g = inputs_label.reshape(_NB // 2, 2, _BLK)
    d1 = _tc_writer(labs2g, out_buf)
    out = _finalize(out_buf, d1, d2)
    return out[None]


# speedup vs baseline: 1.0360x; 1.0360x over previous
import functools

import jax
import jax.numpy as jnp
from jax import lax
from jax.experimental import pallas as pl
from jax.experimental.pallas import tpu as pltpu
from jax.experimental.pallas import tpu_sc as plsc

_DIM = 2048
_C = 7
_SEG = _DIM // _C
_ROWS = 16384
_NC = 2
_NS = 16
_NW = _NC * _NS

_SC_ROWS = 6144
_RPT0 = 224
_RPT1 = 160
_BLK = 512
_NB = _ROWS // _BLK
_TC_OFF = _SC_ROWS // _BLK
_NBT = _NB - _TC_OFF

_mesh = plsc.VectorSubcoreMesh(core_axis_name="c", subcore_axis_name="s")


def _alloc_body(dummy_ref, big_ref):
    dummy_ref[...] = jnp.zeros_like(dummy_ref)


def _alloc_out():
    dummy, big = pl.pallas_call(
        _alloc_body,
        out_shape=(
            jax.ShapeDtypeStruct((8, 128), jnp.float32),
            jax.ShapeDtypeStruct((_ROWS, _DIM), jnp.float32),
        ),
        out_specs=(
            pl.BlockSpec(memory_space=pltpu.VMEM),
            pl.BlockSpec(memory_space=pl.ANY),
        ),
    )()
    return big


@functools.partial(
    pl.kernel,
    mesh=_mesh,
    out_type=jax.ShapeDtypeStruct((16,), jnp.int32),
    scratch_types=[
        pltpu.VMEM((_RPT0,), jnp.int32),
        pltpu.VMEM((8, _DIM), jnp.float32),
        pltpu.SemaphoreType.DMA,
        pltpu.SemaphoreType.DMA,
    ],
)
def _sc_writer(lab_hbm, tab_hbm, out_hbm, done_hbm, lab_v, tab_v, sem_in, sem_out):
    c = lax.axis_index("c")
    s = lax.axis_index("s")
    rpt = jnp.where(c == 0, _RPT0, _RPT1)
    base = s * rpt + c * (_NS * _RPT0)
    pltpu.async_copy(lab_hbm.at[pl.ds(base, _RPT0)], lab_v, sem_in).wait()
    pltpu.async_copy(tab_hbm, tab_v, sem_in).wait()

    @pl.loop(0, _RPT0, step=16)
    def _fire(i):
        @pl.when(i < rpt)
        def _():
            labs = lab_v[pl.ds(i, 16)]
            for k in range(16):
                pltpu.make_async_copy(
                    tab_v.at[labs[k]], out_hbm.at[base + i + k], sem_out
                ).start()

    @pl.loop(0, _RPT0)
    def _drain(i):
        @pl.when(i < rpt)
        def _():
            pltpu.make_async_copy(
                tab_hbm.at[0], out_hbm.at[base + i], sem_out
            ).wait()

    @pl.when(s + c == 0)
    def _():
        pltpu.async_copy(lab_v.at[pl.ds(0, 16)], done_hbm, sem_in).wait()


def _tc_inner(lab_ref, out_ref):
    lab = lab_ref[0, 0, :].reshape(_BLK, 1)
    start = lab * _SEG
    end = jnp.where(lab == _C - 1, _DIM, start + _SEG)
    j = jax.lax.broadcasted_iota(jnp.int32, (_BLK, _DIM), 1)
    mask = (j >= start) & (j < end)
    out_ref[...] = mask.astype(jnp.float32)


def _tc_writer_body(lab_hbm, out_hbm, done_ref):
    done_ref[...] = jnp.zeros_like(done_ref)
    pltpu.emit_pipeline(
        _tc_inner,
        grid=(_NBT,),
        in_specs=[
            pl.BlockSpec((1, 1, _BLK), lambda i: (_TC_OFF + i, 0, 0)),
        ],
        out_specs=[
            pl.BlockSpec((_BLK, _DIM), lambda i: (_TC_OFF + i, 0)),
        ],
    )(lab_hbm, out_hbm)


def _tc_writer(labs3d, out_buf):
    return pl.pallas_call(
        _tc_writer_body,
        in_specs=[
            pl.BlockSpec(memory_space=pl.ANY),
            pl.BlockSpec(memory_space=pl.ANY),
        ],
        out_shape=jax.ShapeDtypeStruct((8, 128), jnp.float32),
        out_specs=pl.BlockSpec(memory_space=pltpu.VMEM),
    )(labs3d, out_buf)


def _final_body(big_ref, d1_ref, d2_ref, out_ref):
    pass


def _finalize(big, d1, d2):
    return pl.pallas_call(
        _final_body,
        in_specs=[
            pl.BlockSpec(memory_space=pl.ANY),
            pl.BlockSpec(memory_space=pltpu.VMEM),
            pl.BlockSpec(memory_space=pltpu.SMEM),
        ],
        out_shape=jax.ShapeDtypeStruct((_ROWS, _DIM), jnp.float32),
        out_specs=pl.BlockSpec(memory_space=pl.ANY),
        input_output_aliases={0: 0},
    )(big, d1, d2)


def kernel(inputs_label):
    r = jnp.arange(8, dtype=jnp.int32)[:, None]
    j = jnp.arange(_DIM, dtype=jnp.int32)[None, :]
    start = r * _SEG
    end = jnp.where(r >= _C - 1, _DIM, start + _SEG)
    tab = ((j >= start) & (j < end)).astype(jnp.float32)

    out_buf = _alloc_out()
    d2 = _sc_writer(inputs_label, tab, out_buf)
    labs3d = inputs_label.reshape(_NB, 1, _BLK)
    d1 = _tc_writer(labs3d, out_buf)
    out = _finalize(out_buf, d1, d2)
    return out[None]


# speedup vs baseline: 1.0688x; 1.0317x over previous
import functools

import jax
import jax.numpy as jnp
from jax import lax
from jax.experimental import pallas as pl
from jax.experimental.pallas import tpu as pltpu
from jax.experimental.pallas import tpu_sc as plsc

_DIM = 2048
_C = 7
_SEG = _DIM // _C
_ROWS = 16384
_NC = 2
_NS = 16
_NW = _NC * _NS

_SC_ROWS = 6144
_RPT = _SC_ROWS // _NW
_BLK = 512
_NB = _ROWS // _BLK
_TC_OFF = _SC_ROWS // _BLK
_NBT = _NB - _TC_OFF

_mesh = plsc.VectorSubcoreMesh(core_axis_name="c", subcore_axis_name="s")


def _alloc_body(dummy_ref, big_ref):
    dummy_ref[...] = jnp.zeros_like(dummy_ref)


def _alloc_out():
    dummy, big = pl.pallas_call(
        _alloc_body,
        out_shape=(
            jax.ShapeDtypeStruct((8, 128), jnp.float32),
            jax.ShapeDtypeStruct((_ROWS, _DIM), jnp.float32),
        ),
        out_specs=(
            pl.BlockSpec(memory_space=pltpu.VMEM),
            pl.BlockSpec(memory_space=pl.ANY),
        ),
    )()
    return big


@functools.partial(
    pl.kernel,
    mesh=_mesh,
    out_type=jax.ShapeDtypeStruct((16,), jnp.int32),
    scratch_types=[
        pltpu.VMEM((_RPT,), jnp.int32),
        pltpu.VMEM((8, _DIM), jnp.float32),
        pltpu.SemaphoreType.DMA,
        pltpu.SemaphoreType.DMA,
    ],
)
def _sc_writer(lab_hbm, tab_hbm, out_hbm, done_hbm, lab_v, tab_v, sem_in, sem_out):
    wid = lax.axis_index("s") * _NC + lax.axis_index("c")
    base = wid * _RPT
    pltpu.async_copy(lab_hbm.at[pl.ds(base, _RPT)], lab_v, sem_in).wait()
    pltpu.async_copy(tab_hbm, tab_v, sem_in).wait()

    @pl.loop(0, _RPT, step=16)
    def _fire(i):
        labs = lab_v[pl.ds(i, 16)]
        for k in range(16):
            pltpu.make_async_copy(
                tab_v.at[labs[k]], out_hbm.at[base + i + k], sem_out
            ).start()

    @pl.loop(0, _RPT)
    def _drain(i):
        pltpu.make_async_copy(
            tab_hbm.at[0], out_hbm.at[base + i], sem_out
        ).wait()

    @pl.when(wid == 0)
    def _():
        pltpu.async_copy(lab_v.at[pl.ds(0, 16)], done_hbm, sem_in).wait()


def _tc_inner(lab_ref, out_ref):
    lab = lab_ref[0, 0, :].reshape(_BLK, 1)
    start = lab * _SEG
    end = jnp.where(lab == _C - 1, _DIM, start + _SEG)
    j = jax.lax.broadcasted_iota(jnp.int32, (_BLK, _DIM), 1)
    mask = (j >= start) & (j < end)
    out_ref[...] = mask.astype(jnp.float32)


def _tc_writer_body(lab_hbm, out_hbm, done_ref):
    done_ref[...] = jnp.zeros_like(done_ref)
    pltpu.emit_pipeline(
        _tc_inner,
        grid=(_NBT,),
        in_specs=[
            pl.BlockSpec((1, 1, _BLK), lambda i: (_TC_OFF + i, 0, 0)),
        ],
        out_specs=[
            pl.BlockSpec((_BLK, _DIM), lambda i: (_TC_OFF + i, 0)),
        ],
    )(lab_hbm, out_hbm)


def _tc_writer(labs3d, out_buf):
    return pl.pallas_call(
        _tc_writer_body,
        in_specs=[
            pl.BlockSpec(memory_space=pl.ANY),
            pl.BlockSpec(memory_space=pl.ANY),
        ],
        out_shape=jax.ShapeDtypeStruct((8, 128), jnp.float32),
        out_specs=pl.BlockSpec(memory_space=pltpu.VMEM),
    )(labs3d, out_buf)


def _final_body(big_ref, d1_ref, d2_ref, out_ref):
    pass


def _finalize(big, d1, d2):
    return pl.pallas_call(
        _final_body,
        in_specs=[
            pl.BlockSpec(memory_space=pl.ANY),
            pl.BlockSpec(memory_space=pltpu.VMEM),
            pl.BlockSpec(memory_space=pltpu.SMEM),
        ],
        out_shape=jax.ShapeDtypeStruct((_ROWS, _DIM), jnp.float32),
        out_specs=pl.BlockSpec(memory_space=pl.ANY),
        input_output_aliases={0: 0},
    )(big, d1, d2)


def kernel(inputs_label):
    r = jnp.arange(8, dtype=jnp.int32)[:, None]
    j = jnp.arange(_DIM, dtype=jnp.int32)[None, :]
    start = r * _SEG
    end = jnp.where(r >= _C - 1, _DIM, start + _SEG)
    tab = ((j >= start) & (j < end)).astype(jnp.float32)

    out_buf = _alloc_out()
    d2 = _sc_writer(inputs_label, tab, out_buf)
    labs3d = inputs_label.reshape(_NB, 1, _BLK)
    d1 = _tc_writer(labs3d, out_buf)
    out = _finalize(out_buf, d1, d2)
    return out[None]


# speedup vs baseline: 1.1122x; 1.0406x over previous
import functools

import jax
import jax.numpy as jnp
from jax import lax
from jax.experimental import pallas as pl
from jax.experimental.pallas import tpu as pltpu
from jax.experimental.pallas import tpu_sc as plsc

_DIM = 2048
_C = 7
_SEG = _DIM // _C
_ROWS = 16384
_NC = 2
_NS = 16
_NW = _NC * _NS

_SC_ROWS = 5120
_RPT = _SC_ROWS // _NW
_BLK = 512
_NB = _ROWS // _BLK
_TC_OFF = _SC_ROWS // _BLK
_NBT = _NB - _TC_OFF

_mesh = plsc.VectorSubcoreMesh(core_axis_name="c", subcore_axis_name="s")


def _alloc_body(dummy_ref, big_ref):
    dummy_ref[...] = jnp.zeros_like(dummy_ref)


def _alloc_out():
    dummy, big = pl.pallas_call(
        _alloc_body,
        out_shape=(
            jax.ShapeDtypeStruct((8, 128), jnp.float32),
            jax.ShapeDtypeStruct((_ROWS, _DIM), jnp.float32),
        ),
        out_specs=(
            pl.BlockSpec(memory_space=pltpu.VMEM),
            pl.BlockSpec(memory_space=pl.ANY),
        ),
    )()
    return big


@functools.partial(
    pl.kernel,
    mesh=_mesh,
    out_type=jax.ShapeDtypeStruct((16,), jnp.int32),
    scratch_types=[
        pltpu.VMEM((_RPT,), jnp.int32),
        pltpu.VMEM((8, _DIM), jnp.float32),
        pltpu.SemaphoreType.DMA,
        pltpu.SemaphoreType.DMA,
    ],
)
def _sc_writer(lab_hbm, tab_hbm, out_hbm, done_hbm, lab_v, tab_v, sem_in, sem_out):
    wid = lax.axis_index("s") * _NC + lax.axis_index("c")
    base = wid * _RPT
    pltpu.async_copy(lab_hbm.at[pl.ds(base, _RPT)], lab_v, sem_in).wait()
    pltpu.async_copy(tab_hbm, tab_v, sem_in).wait()

    @pl.loop(0, _RPT, step=16)
    def _fire(i):
        labs = lab_v[pl.ds(i, 16)]
        for k in range(16):
            pltpu.make_async_copy(
                tab_v.at[labs[k]], out_hbm.at[base + i + k], sem_out
            ).start()

    @pl.loop(0, _RPT)
    def _drain(i):
        pltpu.make_async_copy(
            tab_hbm.at[0], out_hbm.at[base + i], sem_out
        ).wait()

    @pl.when(wid == 0)
    def _():
        pltpu.async_copy(lab_v.at[pl.ds(0, 16)], done_hbm, sem_in).wait()


def _tc_inner(lab_ref, out_ref):
    lab = lab_ref[0, 0, :].reshape(_BLK, 1)
    start = lab * _SEG
    end = jnp.where(lab == _C - 1, _DIM, start + _SEG)
    j = jax.lax.broadcasted_iota(jnp.int32, (_BLK, _DIM), 1)
    mask = (j >= start) & (j < end)
    out_ref[...] = mask.astype(jnp.float32)


def _tc_writer_body(lab_hbm, out_hbm, done_ref):
    done_ref[...] = jnp.zeros_like(done_ref)
    pltpu.emit_pipeline(
        _tc_inner,
        grid=(_NBT,),
        in_specs=[
            pl.BlockSpec((1, 1, _BLK), lambda i: (_TC_OFF + i, 0, 0)),
        ],
        out_specs=[
            pl.BlockSpec((_BLK, _DIM), lambda i: (_TC_OFF + i, 0)),
        ],
    )(lab_hbm, out_hbm)


def _tc_writer(labs3d, out_buf):
    return pl.pallas_call(
        _tc_writer_body,
        in_specs=[
            pl.BlockSpec(memory_space=pl.ANY),
            pl.BlockSpec(memory_space=pl.ANY),
        ],
        out_shape=jax.ShapeDtypeStruct((8, 128), jnp.float32),
        out_specs=pl.BlockSpec(memory_space=pltpu.VMEM),
    )(labs3d, out_buf)


def _final_body(big_ref, d1_ref, d2_ref, out_ref):
    pass


def _finalize(big, d1, d2):
    return pl.pallas_call(
        _final_body,
        in_specs=[
            pl.BlockSpec(memory_space=pl.ANY),
            pl.BlockSpec(memory_space=pltpu.VMEM),
            pl.BlockSpec(memory_space=pltpu.SMEM),
        ],
        out_shape=jax.ShapeDtypeStruct((_ROWS, _DIM), jnp.float32),
        out_specs=pl.BlockSpec(memory_space=pl.ANY),
        input_output_aliases={0: 0},
    )(big, d1, d2)


def kernel(inputs_label):
    r = jnp.arange(8, dtype=jnp.int32)[:, None]
    j = jnp.arange(_DIM, dtype=jnp.int32)[None, :]
    start = r * _SEG
    end = jnp.where(r >= _C - 1, _DIM, start + _SEG)
    tab = ((j >= start) & (j < end)).astype(jnp.float32)

    out_buf = _alloc_out()
    d2 = _sc_writer(inputs_label, tab, out_buf)
    labs3d = inputs_label.reshape(_NB, 1, _BLK)
    d1 = _tc_writer(labs3d, out_buf)
    out = _finalize(out_buf, d1, d2)
    return out[None]
